# Initial kernel scaffold; baseline (speedup 1.0000x reference)
import jax
import jax.numpy as jnp
from jax.experimental import pallas as pl


def kernel(x, edge_index, batch, metadata, W1, b1, W2, b2, W3, b3, Wh1, bh1, Wh2, bh2):
    raise NotImplementedError("write your pallas kernel here")


# speedup vs baseline: 25.4289x; 25.4289x over previous
import jax
import jax.numpy as jnp
from jax import lax
from jax.experimental import pallas as pl
from jax.experimental.pallas import tpu as pltpu
from jax.experimental.pallas import tpu_sc as plsc

_N = 10000
_E = 320000
_HID = 64
_NC, _NS = 2, 16
_NW = _NC * _NS
_CH = 125
_NCHUNK = _E // (_NW * _CH)
_NPAD = 10240
_RPS = _NPAD // _NS
_DW = 8
_BLK = 2000


import functools as _functools


@_functools.lru_cache(maxsize=None)
def _sc_mesh():
    return plsc.VectorSubcoreMesh(
        core_axis_name="c", subcore_axis_name="s",
        num_cores=_NC, num_subcores=_NS)

def _deg_body(dst_hbm, ones_hbm, zeros_hbm, out_hbm, dst_v, ones_v, acc_sh):
    c = lax.axis_index("c")
    s = lax.axis_index("s")
    wid = c * _NS + s
    r0 = s * _RPS
    pltpu.sync_copy(zeros_hbm, acc_sh.at[pl.ds(r0, _RPS)])
    pltpu.sync_copy(ones_hbm, ones_v)
    pltpu.sync_copy(dst_hbm.at[pl.ds(wid * _NCHUNK, _NCHUNK)], dst_v)
    plsc.subcore_barrier()

    @pl.loop(0, _NCHUNK)
    def _(j):
        pltpu.sync_copy(ones_v, acc_sh.at[dst_v.at[j]], add=True)

    plsc.subcore_barrier()
    pltpu.sync_copy(acc_sh.at[pl.ds(r0, _RPS)],
                    out_hbm.at[pl.ds(c * _NPAD + r0, _RPS)])


def _sc_deg(dst2, ones_d, zeros_d):
    return pl.kernel(
        _deg_body,
        out_type=jax.ShapeDtypeStruct((_NC * _NPAD, _DW), jnp.float32),
        mesh=_sc_mesh(),
        scratch_types=[
            pltpu.VMEM((_NCHUNK, _CH), jnp.int32),
            pltpu.VMEM((_CH, _DW), jnp.float32),
            pltpu.VMEM_SHARED((_NPAD, _DW), jnp.float32),
        ],
        compiler_params=pltpu.CompilerParams(use_tc_tiling_on_sc=False),
    )(dst2, ones_d, zeros_d)


def _agg_body(src_hbm, dst_hbm, table_hbm, zeros_hbm, out_hbm,
              src_v, dst_v, rows_v, table_sh, acc_sh):
    c = lax.axis_index("c")
    s = lax.axis_index("s")
    wid = c * _NS + s
    r0 = s * _RPS
    pltpu.sync_copy(zeros_hbm, acc_sh.at[pl.ds(r0, _RPS)])
    t0 = s * 624
    pltpu.sync_copy(table_hbm.at[pl.ds(t0, 624)], table_sh.at[pl.ds(t0, 624)])

    @pl.when(s == _NS - 1)
    def _():
        pltpu.sync_copy(table_hbm.at[pl.ds(9984, 16)],
                        table_sh.at[pl.ds(9984, 16)])

    pltpu.sync_copy(src_hbm.at[pl.ds(wid * _NCHUNK, _NCHUNK)], src_v)
    pltpu.sync_copy(dst_hbm.at[pl.ds(wid * _NCHUNK, _NCHUNK)], dst_v)
    plsc.subcore_barrier()

    @pl.loop(0, _NCHUNK)
    def _(j):
        pltpu.sync_copy(table_sh.at[src_v.at[j]], rows_v)
        pltpu.sync_copy(rows_v, acc_sh.at[dst_v.at[j]], add=True)

    plsc.subcore_barrier()
    pltpu.sync_copy(acc_sh.at[pl.ds(r0, _RPS)],
                    out_hbm.at[pl.ds(c * _NPAD + r0, _RPS)])


def _sc_agg(src2, dst2, table, zeros_h):
    return pl.kernel(
        _agg_body,
        out_type=jax.ShapeDtypeStruct((_NC * _NPAD, _HID), jnp.float32),
        mesh=_sc_mesh(),
        scratch_types=[
            pltpu.VMEM((_NCHUNK, _CH), jnp.int32),
            pltpu.VMEM((_NCHUNK, _CH), jnp.int32),
            pltpu.VMEM((_CH, _HID), jnp.float32),
            pltpu.VMEM_SHARED((_N, _HID), jnp.float32),
            pltpu.VMEM_SHARED((_NPAD, _HID), jnp.float32),
        ],
        compiler_params=pltpu.CompilerParams(use_tc_tiling_on_sc=False),
    )(src2, dst2, table, zeros_h)


def _dinv_of(deg_ref):
    deg = 1.0 + deg_ref[:, 0:1] + deg_ref[:, 1:2]
    return lax.rsqrt(deg)


def _tc_in_body(x_ref, w_ref, deg_ref, y_ref):
    dinv = _dinv_of(deg_ref)
    y_ref[...] = jnp.dot(x_ref[...], w_ref[...],
                         preferred_element_type=jnp.float32) * dinv


def _tc_mid_body(y_ref, sp_ref, deg_ref, w_ref, b_ref, o_ref):
    dinv = _dinv_of(deg_ref)
    stot = y_ref[...] + sp_ref[0] + sp_ref[1]
    h = jnp.maximum(stot * dinv + b_ref[...], 0.0)
    o_ref[...] = jnp.dot(h, w_ref[...],
                         preferred_element_type=jnp.float32) * dinv


def _tc_head_body(y_ref, sp_ref, deg_ref, b_ref, batch_ref, md_ref,
                  wh1a_ref, wh1b_ref, bh1_ref, wh2_ref, bh2_ref,
                  out_ref, sums_sc, cnts_sc):
    i = pl.program_id(0)

    @pl.when(i == 0)
    def _():
        sums_sc[...] = jnp.zeros_like(sums_sc)
        cnts_sc[...] = jnp.zeros_like(cnts_sc)

    dinv = _dinv_of(deg_ref)
    stot = y_ref[...] + sp_ref[0] + sp_ref[1]
    h = jnp.maximum(stot * dinv + b_ref[...], 0.0)
    bt = batch_ref[0, 0]
    seg = lax.broadcasted_iota(jnp.int32, (64, _BLK), 0)
    mask = (seg == jnp.broadcast_to(bt[None, :], (64, _BLK))).astype(jnp.float32)
    sums_sc[...] += jnp.dot(mask, h, preferred_element_type=jnp.float32)
    cnts_sc[...] += jnp.sum(mask, axis=1, keepdims=True)

    @pl.when(i == pl.num_programs(0) - 1)
    def _():
        emb = sums_sc[...] / jnp.maximum(cnts_sc[...], 1.0)
        hh = (jnp.dot(emb, wh1a_ref[...], preferred_element_type=jnp.float32)
              + jnp.dot(md_ref[...], wh1b_ref[...],
                        preferred_element_type=jnp.float32)
              + bh1_ref[...])
        hh = jnp.maximum(hh, 0.0)
        out_ref[...] = jnp.dot(hh, wh2_ref[...],
                               preferred_element_type=jnp.float32) + bh2_ref[...]


def _tc_in(x, W, degt):
    return pl.pallas_call(
        _tc_in_body,
        grid=(_N // _BLK,),
        in_specs=[
            pl.BlockSpec((_BLK, 128), lambda i: (i, 0)),
            pl.BlockSpec((128, _HID), lambda i: (0, 0)),
            pl.BlockSpec((_BLK, 2), lambda i: (i, 0)),
        ],
        out_specs=pl.BlockSpec((_BLK, _HID), lambda i: (i, 0)),
        out_shape=jax.ShapeDtypeStruct((_N, _HID), jnp.float32),
    )(x, W, degt)


def _tc_mid(y, sp, degt, W, b):
    return pl.pallas_call(
        _tc_mid_body,
        grid=(_N // _BLK,),
        in_specs=[
            pl.BlockSpec((_BLK, _HID), lambda i: (i, 0)),
            pl.BlockSpec((_NC, _BLK, _HID), lambda i: (0, i, 0)),
            pl.BlockSpec((_BLK, 2), lambda i: (i, 0)),
            pl.BlockSpec((_HID, _HID), lambda i: (0, 0)),
            pl.BlockSpec((_HID,), lambda i: (0,)),
        ],
        out_specs=pl.BlockSpec((_BLK, _HID), lambda i: (i, 0)),
        out_shape=jax.ShapeDtypeStruct((_N, _HID), jnp.float32),
    )(y, sp, degt, W, b)


def _tc_head(y, sp, degt, b, batch, md_pad, Wh1a, Wh1b_pad, bh1, Wh2, bh2):
    return pl.pallas_call(
        _tc_head_body,
        grid=(_N // _BLK,),
        in_specs=[
            pl.BlockSpec((_BLK, _HID), lambda i: (i, 0)),
            pl.BlockSpec((_NC, _BLK, _HID), lambda i: (0, i, 0)),
            pl.BlockSpec((_BLK, 2), lambda i: (i, 0)),
            pl.BlockSpec((_HID,), lambda i: (0,)),
            pl.BlockSpec((1, 1, _BLK), lambda i: (i, 0, 0)),
            pl.BlockSpec((64, 32), lambda i: (0, 0)),
            pl.BlockSpec((_HID, _HID), lambda i: (0, 0)),
            pl.BlockSpec((32, _HID), lambda i: (0, 0)),
            pl.BlockSpec((_HID,), lambda i: (0,)),
            pl.BlockSpec((_HID, 1), lambda i: (0, 0)),
            pl.BlockSpec((1, 1), lambda i: (0, 0)),
        ],
        out_specs=pl.BlockSpec((64, 1), lambda i: (0, 0)),
        out_shape=jax.ShapeDtypeStruct((64, 1), jnp.float32),
        scratch_shapes=[
            pltpu.VMEM((64, _HID), jnp.float32),
            pltpu.VMEM((64, 1), jnp.float32),
        ],
    )(y, sp, degt, b, batch, md_pad, Wh1a, Wh1b_pad, bh1, Wh2, bh2)


def kernel(x, edge_index, batch, metadata, W1, b1, W2, b2, W3, b3,
           Wh1, bh1, Wh2, bh2):
    src2 = edge_index[0].reshape(_NW * _NCHUNK, _CH)
    dst2 = edge_index[1].reshape(_NW * _NCHUNK, _CH)

    ones_d = jnp.ones((_CH, _DW), jnp.float32)
    zeros_d = jnp.zeros((_RPS, _DW), jnp.float32)
    zeros_h = jnp.zeros((_RPS, _HID), jnp.float32)

    degp = _sc_deg(dst2, ones_d, zeros_d)
    degt = degp[:, 0].reshape(_NC, _NPAD)[:, :_N].T

    y1 = _tc_in(x, W1, degt)
    s1 = _sc_agg(src2, dst2, y1, zeros_h).reshape(_NC, _NPAD, _HID)
    y2 = _tc_mid(y1, s1, degt, W2, b1)
    s2 = _sc_agg(src2, dst2, y2, zeros_h).reshape(_NC, _NPAD, _HID)
    y3 = _tc_mid(y2, s2, degt, W3, b2)
    s3 = _sc_agg(src2, dst2, y3, zeros_h).reshape(_NC, _NPAD, _HID)

    md_pad = jnp.pad(metadata, ((0, 0), (0, 32 - metadata.shape[1])))
    Wh1a = Wh1[:_HID]
    Wh1b_pad = jnp.pad(Wh1[_HID:], ((0, 32 - (Wh1.shape[0] - _HID)), (0, 0)))
    bh2r = bh2.reshape(1, 1)

    batch3 = batch.reshape(_N // _BLK, 1, _BLK)
    return _tc_head(y3, s3, degt, b3, batch3, md_pad, Wh1a, Wh1b_pad,
                    bh1, Wh2, bh2r)


# speedup vs baseline: 28.6464x; 1.1265x over previous
import jax
import jax.numpy as jnp
from jax import lax
from jax.experimental import pallas as pl
from jax.experimental.pallas import tpu as pltpu
from jax.experimental.pallas import tpu_sc as plsc

_N = 10000
_E = 320000
_HID = 64
_NC, _NS = 2, 16
_NW = _NC * _NS
_CH = 125
_NCHUNK = _E // (_NW * _CH)
_NPAD = 10240
_RPS = _NPAD // _NS
_DW = 8
_BLK = 2000


import functools as _functools


@_functools.lru_cache(maxsize=None)
def _sc_mesh():
    return plsc.VectorSubcoreMesh(
        core_axis_name="c", subcore_axis_name="s",
        num_cores=_NC, num_subcores=_NS)

def _deg_body(dst_hbm, ones_hbm, zeros_hbm, out_hbm, dst_v, ones_v, acc_sh):
    c = lax.axis_index("c")
    s = lax.axis_index("s")
    wid = c * _NS + s
    r0 = s * _RPS
    pltpu.sync_copy(zeros_hbm, acc_sh.at[pl.ds(r0, _RPS)])
    pltpu.sync_copy(ones_hbm, ones_v)
    pltpu.sync_copy(dst_hbm.at[pl.ds(wid * _NCHUNK, _NCHUNK)], dst_v)
    plsc.subcore_barrier()

    @pl.loop(0, _NCHUNK)
    def _(j):
        pltpu.sync_copy(ones_v, acc_sh.at[dst_v.at[j]], add=True)

    plsc.subcore_barrier()
    pltpu.sync_copy(acc_sh.at[pl.ds(r0, _RPS)],
                    out_hbm.at[pl.ds(c * _NPAD + r0, _RPS)])


def _sc_deg(dst2, ones_d, zeros_d):
    return pl.kernel(
        _deg_body,
        out_type=jax.ShapeDtypeStruct((_NC * _NPAD, _DW), jnp.float32),
        mesh=_sc_mesh(),
        scratch_types=[
            pltpu.VMEM((_NCHUNK, _CH), jnp.int32),
            pltpu.VMEM((_CH, _DW), jnp.float32),
            pltpu.VMEM_SHARED((_NPAD, _DW), jnp.float32),
        ],
        compiler_params=pltpu.CompilerParams(use_tc_tiling_on_sc=False),
    )(dst2, ones_d, zeros_d)


def _agg_body(src_hbm, dst_hbm, table_hbm, zeros_hbm, out_hbm,
              src_v, dst_v, rows0_v, rows1_v, acc_sh, sem0, sem1):
    c = lax.axis_index("c")
    s = lax.axis_index("s")
    wid = c * _NS + s
    r0 = s * _RPS
    pltpu.sync_copy(zeros_hbm, acc_sh.at[pl.ds(r0, _RPS)])
    pltpu.sync_copy(src_hbm.at[pl.ds(wid * _NCHUNK, _NCHUNK)], src_v)
    pltpu.sync_copy(dst_hbm.at[pl.ds(wid * _NCHUNK, _NCHUNK)], dst_v)
    plsc.subcore_barrier()

    pltpu.async_copy(table_hbm.at[src_v.at[0]], rows0_v, sem0)

    @pl.loop(0, _NCHUNK, step=2)
    def _(j):
        pltpu.make_async_copy(table_hbm.at[src_v.at[j]], rows0_v, sem0).wait()
        pltpu.async_copy(table_hbm.at[src_v.at[j + 1]], rows1_v, sem1)
        pltpu.sync_copy(rows0_v, acc_sh.at[dst_v.at[j]], add=True)
        pltpu.make_async_copy(table_hbm.at[src_v.at[j + 1]], rows1_v,
                              sem1).wait()

        @pl.when(j + 2 < _NCHUNK)
        def _():
            pltpu.async_copy(table_hbm.at[src_v.at[j + 2]], rows0_v, sem0)

        pltpu.sync_copy(rows1_v, acc_sh.at[dst_v.at[j + 1]], add=True)

    plsc.subcore_barrier()
    pltpu.sync_copy(acc_sh.at[pl.ds(r0, _RPS)],
                    out_hbm.at[pl.ds(c * _NPAD + r0, _RPS)])


def _sc_agg(src2, dst2, table, zeros_h):
    return pl.kernel(
        _agg_body,
        out_type=jax.ShapeDtypeStruct((_NC * _NPAD, _HID), jnp.float32),
        mesh=_sc_mesh(),
        scratch_types=[
            pltpu.VMEM((_NCHUNK, _CH), jnp.int32),
            pltpu.VMEM((_NCHUNK, _CH), jnp.int32),
            pltpu.VMEM((_CH, _HID), jnp.float32),
            pltpu.VMEM((_CH, _HID), jnp.float32),
            pltpu.VMEM_SHARED((_NPAD, _HID), jnp.float32),
            pltpu.SemaphoreType.DMA,
            pltpu.SemaphoreType.DMA,
        ],
        compiler_params=pltpu.CompilerParams(use_tc_tiling_on_sc=False),
    )(src2, dst2, table, zeros_h)


def _dinv_of(deg_ref):
    deg = 1.0 + deg_ref[:, 0:1] + deg_ref[:, 1:2]
    return lax.rsqrt(deg)


def _tc_in_body(x_ref, w_ref, deg_ref, y_ref):
    dinv = _dinv_of(deg_ref)
    y_ref[...] = jnp.dot(x_ref[...], w_ref[...],
                         preferred_element_type=jnp.float32) * dinv


def _tc_mid_body(y_ref, sp_ref, deg_ref, w_ref, b_ref, o_ref):
    dinv = _dinv_of(deg_ref)
    stot = y_ref[...] + sp_ref[0] + sp_ref[1]
    h = jnp.maximum(stot * dinv + b_ref[...], 0.0)
    o_ref[...] = jnp.dot(h, w_ref[...],
                         preferred_element_type=jnp.float32) * dinv


def _tc_head_body(y_ref, sp_ref, deg_ref, b_ref, batch_ref, md_ref,
                  wh1a_ref, wh1b_ref, bh1_ref, wh2_ref, bh2_ref,
                  out_ref, sums_sc, cnts_sc):
    i = pl.program_id(0)

    @pl.when(i == 0)
    def _():
        sums_sc[...] = jnp.zeros_like(sums_sc)
        cnts_sc[...] = jnp.zeros_like(cnts_sc)

    dinv = _dinv_of(deg_ref)
    stot = y_ref[...] + sp_ref[0] + sp_ref[1]
    h = jnp.maximum(stot * dinv + b_ref[...], 0.0)
    bt = batch_ref[0, 0]
    seg = lax.broadcasted_iota(jnp.int32, (64, _BLK), 0)
    mask = (seg == jnp.broadcast_to(bt[None, :], (64, _BLK))).astype(jnp.float32)
    sums_sc[...] += jnp.dot(mask, h, preferred_element_type=jnp.float32)
    cnts_sc[...] += jnp.sum(mask, axis=1, keepdims=True)

    @pl.when(i == pl.num_programs(0) - 1)
    def _():
        emb = sums_sc[...] / jnp.maximum(cnts_sc[...], 1.0)
        hh = (jnp.dot(emb, wh1a_ref[...], preferred_element_type=jnp.float32)
              + jnp.dot(md_ref[...], wh1b_ref[...],
                        preferred_element_type=jnp.float32)
              + bh1_ref[...])
        hh = jnp.maximum(hh, 0.0)
        out_ref[...] = jnp.dot(hh, wh2_ref[...],
                               preferred_element_type=jnp.float32) + bh2_ref[...]


def _tc_in(x, W, degt):
    return pl.pallas_call(
        _tc_in_body,
        grid=(_N // _BLK,),
        in_specs=[
            pl.BlockSpec((_BLK, 128), lambda i: (i, 0)),
            pl.BlockSpec((128, _HID), lambda i: (0, 0)),
            pl.BlockSpec((_BLK, 2), lambda i: (i, 0)),
        ],
        out_specs=pl.BlockSpec((_BLK, _HID), lambda i: (i, 0)),
        out_shape=jax.ShapeDtypeStruct((_N, _HID), jnp.float32),
    )(x, W, degt)


def _tc_mid(y, sp, degt, W, b):
    return pl.pallas_call(
        _tc_mid_body,
        grid=(_N // _BLK,),
        in_specs=[
            pl.BlockSpec((_BLK, _HID), lambda i: (i, 0)),
            pl.BlockSpec((_NC, _BLK, _HID), lambda i: (0, i, 0)),
            pl.BlockSpec((_BLK, 2), lambda i: (i, 0)),
            pl.BlockSpec((_HID, _HID), lambda i: (0, 0)),
            pl.BlockSpec((_HID,), lambda i: (0,)),
        ],
        out_specs=pl.BlockSpec((_BLK, _HID), lambda i: (i, 0)),
        out_shape=jax.ShapeDtypeStruct((_N, _HID), jnp.float32),
    )(y, sp, degt, W, b)


def _tc_head(y, sp, degt, b, batch, md_pad, Wh1a, Wh1b_pad, bh1, Wh2, bh2):
    return pl.pallas_call(
        _tc_head_body,
        grid=(_N // _BLK,),
        in_specs=[
            pl.BlockSpec((_BLK, _HID), lambda i: (i, 0)),
            pl.BlockSpec((_NC, _BLK, _HID), lambda i: (0, i, 0)),
            pl.BlockSpec((_BLK, 2), lambda i: (i, 0)),
            pl.BlockSpec((_HID,), lambda i: (0,)),
            pl.BlockSpec((1, 1, _BLK), lambda i: (i, 0, 0)),
            pl.BlockSpec((64, 32), lambda i: (0, 0)),
            pl.BlockSpec((_HID, _HID), lambda i: (0, 0)),
            pl.BlockSpec((32, _HID), lambda i: (0, 0)),
            pl.BlockSpec((_HID,), lambda i: (0,)),
            pl.BlockSpec((_HID, 1), lambda i: (0, 0)),
            pl.BlockSpec((1, 1), lambda i: (0, 0)),
        ],
        out_specs=pl.BlockSpec((64, 1), lambda i: (0, 0)),
        out_shape=jax.ShapeDtypeStruct((64, 1), jnp.float32),
        scratch_shapes=[
            pltpu.VMEM((64, _HID), jnp.float32),
            pltpu.VMEM((64, 1), jnp.float32),
        ],
    )(y, sp, degt, b, batch, md_pad, Wh1a, Wh1b_pad, bh1, Wh2, bh2)


def kernel(x, edge_index, batch, metadata, W1, b1, W2, b2, W3, b3,
           Wh1, bh1, Wh2, bh2):
    src2 = edge_index[0].reshape(_NW * _NCHUNK, _CH)
    dst2 = edge_index[1].reshape(_NW * _NCHUNK, _CH)

    ones_d = jnp.ones((_CH, _DW), jnp.float32)
    zeros_d = jnp.zeros((_RPS, _DW), jnp.float32)
    zeros_h = jnp.zeros((_RPS, _HID), jnp.float32)

    degp = _sc_deg(dst2, ones_d, zeros_d)
    degt = degp[:, 0].reshape(_NC, _NPAD)[:, :_N].T

    y1 = _tc_in(x, W1, degt)
    s1 = _sc_agg(src2, dst2, y1, zeros_h).reshape(_NC, _NPAD, _HID)
    y2 = _tc_mid(y1, s1, degt, W2, b1)
    s2 = _sc_agg(src2, dst2, y2, zeros_h).reshape(_NC, _NPAD, _HID)
    y3 = _tc_mid(y2, s2, degt, W3, b2)
    s3 = _sc_agg(src2, dst2, y3, zeros_h).reshape(_NC, _NPAD, _HID)

    md_pad = jnp.pad(metadata, ((0, 0), (0, 32 - metadata.shape[1])))
    Wh1a = Wh1[:_HID]
    Wh1b_pad = jnp.pad(Wh1[_HID:], ((0, 32 - (Wh1.shape[0] - _HID)), (0, 0)))
    bh2r = bh2.reshape(1, 1)

    batch3 = batch.reshape(_N // _BLK, 1, _BLK)
    return _tc_head(y3, s3, degt, b3, batch3, md_pad, Wh1a, Wh1b_pad,
                    bh1, Wh2, bh2r)


# speedup vs baseline: 35.8931x; 1.2530x over previous
import jax
import jax.numpy as jnp
from jax import lax
from jax.experimental import pallas as pl
from jax.experimental.pallas import tpu as pltpu
from jax.experimental.pallas import tpu_sc as plsc

_N = 10000
_E = 320000
_HID = 64
_NC, _NS = 2, 16
_NW = _NC * _NS
_CH = 125
_NCHUNK = _E // (_NW * _CH)
_NPAD = 10240
_RPS = _NPAD // _NS
_DW = 8
_BLK = 2000


import functools as _functools


@_functools.lru_cache(maxsize=None)
def _sc_mesh():
    return plsc.VectorSubcoreMesh(
        core_axis_name="c", subcore_axis_name="s",
        num_cores=_NC, num_subcores=_NS)

def _deg_body(dst_hbm, ones_hbm, zeros_hbm, out_hbm, dst_v, ones_v, acc_sh):
    c = lax.axis_index("c")
    s = lax.axis_index("s")
    wid = c * _NS + s
    r0 = s * _RPS
    pltpu.sync_copy(zeros_hbm, acc_sh.at[pl.ds(r0, _RPS)])
    pltpu.sync_copy(ones_hbm, ones_v)
    pltpu.sync_copy(dst_hbm.at[pl.ds(wid * _NCHUNK, _NCHUNK)], dst_v)
    plsc.subcore_barrier()

    @pl.loop(0, _NCHUNK)
    def _(j):
        pltpu.sync_copy(ones_v, acc_sh.at[dst_v.at[j]], add=True)

    plsc.subcore_barrier()
    pltpu.sync_copy(acc_sh.at[pl.ds(r0, _RPS)],
                    out_hbm.at[pl.ds(c * _NPAD + r0, _RPS)])


def _sc_deg(dst2, ones_d, zeros_d):
    return pl.kernel(
        _deg_body,
        out_type=jax.ShapeDtypeStruct((_NC * _NPAD, _DW), jnp.float32),
        mesh=_sc_mesh(),
        scratch_types=[
            pltpu.VMEM((_NCHUNK, _CH), jnp.int32),
            pltpu.VMEM((_CH, _DW), jnp.float32),
            pltpu.VMEM_SHARED((_NPAD, _DW), jnp.float32),
        ],
        compiler_params=pltpu.CompilerParams(use_tc_tiling_on_sc=False),
    )(dst2, ones_d, zeros_d)


_NBUF = 4


def _agg_body(src_hbm, dst_hbm, table_hbm, zeros_hbm, out_hbm,
              src_v, dst_v, rows_v, acc_sh, *sems):
    semg = sems[:_NBUF]
    sems_ = sems[_NBUF:]
    c = lax.axis_index("c")
    s = lax.axis_index("s")
    wid = c * _NS + s
    r0 = s * _RPS
    pltpu.sync_copy(zeros_hbm, acc_sh.at[pl.ds(r0, _RPS)])
    pltpu.sync_copy(src_hbm.at[pl.ds(wid * _NCHUNK, _NCHUNK)], src_v)
    pltpu.sync_copy(dst_hbm.at[pl.ds(wid * _NCHUNK, _NCHUNK)], dst_v)
    plsc.subcore_barrier()

    for b in range(_NBUF):
        pltpu.async_copy(table_hbm.at[src_v.at[b]], rows_v[b], semg[b])

    @pl.loop(0, _NCHUNK, step=_NBUF)
    def _(j):
        for b in range(_NBUF):
            pltpu.make_async_copy(table_hbm.at[src_v.at[j + b]], rows_v[b],
                                  semg[b]).wait()
            pltpu.async_copy(rows_v[b], acc_sh.at[dst_v.at[j + b]], sems_[b],
                             add=True)
        for b in range(_NBUF):
            @pl.when(j + b + _NBUF < _NCHUNK)
            def _(b=b):
                pltpu.make_async_copy(rows_v[b], acc_sh.at[dst_v.at[j + b]],
                                      sems_[b]).wait()
                pltpu.async_copy(table_hbm.at[src_v.at[j + b + _NBUF]],
                                 rows_v[b], semg[b])

    for b in range(_NBUF):
        pltpu.make_async_copy(rows_v[b],
                              acc_sh.at[dst_v.at[_NCHUNK - _NBUF + b]],
                              sems_[b]).wait()

    plsc.subcore_barrier()
    pltpu.sync_copy(acc_sh.at[pl.ds(r0, _RPS)],
                    out_hbm.at[pl.ds(c * _NPAD + r0, _RPS)])


def _sc_agg(src2, dst2, table, zeros_h):
    return pl.kernel(
        _agg_body,
        out_type=jax.ShapeDtypeStruct((_NC * _NPAD, _HID), jnp.float32),
        mesh=_sc_mesh(),
        scratch_types=(
            [pltpu.VMEM((_NCHUNK, _CH), jnp.int32),
             pltpu.VMEM((_NCHUNK, _CH), jnp.int32),
             [pltpu.VMEM((_CH, _HID), jnp.float32) for _ in range(_NBUF)],
             pltpu.VMEM_SHARED((_NPAD, _HID), jnp.float32)]
            + [pltpu.SemaphoreType.DMA] * (2 * _NBUF)
        ),
        compiler_params=pltpu.CompilerParams(use_tc_tiling_on_sc=False),
    )(src2, dst2, table, zeros_h)


def _dinv_of(deg_ref):
    deg = 1.0 + deg_ref[:, 0:1] + deg_ref[:, 1:2]
    return lax.rsqrt(deg)


def _tc_in_body(x_ref, w_ref, deg_ref, y_ref):
    dinv = _dinv_of(deg_ref)
    y_ref[...] = jnp.dot(x_ref[...], w_ref[...],
                         preferred_element_type=jnp.float32) * dinv


def _tc_mid_body(y_ref, sp_ref, deg_ref, w_ref, b_ref, o_ref):
    dinv = _dinv_of(deg_ref)
    stot = y_ref[...] + sp_ref[0] + sp_ref[1]
    h = jnp.maximum(stot * dinv + b_ref[...], 0.0)
    o_ref[...] = jnp.dot(h, w_ref[...],
                         preferred_element_type=jnp.float32) * dinv


def _tc_head_body(y_ref, sp_ref, deg_ref, b_ref, batch_ref, md_ref,
                  wh1a_ref, wh1b_ref, bh1_ref, wh2_ref, bh2_ref,
                  out_ref, sums_sc, cnts_sc):
    i = pl.program_id(0)

    @pl.when(i == 0)
    def _():
        sums_sc[...] = jnp.zeros_like(sums_sc)
        cnts_sc[...] = jnp.zeros_like(cnts_sc)

    dinv = _dinv_of(deg_ref)
    stot = y_ref[...] + sp_ref[0] + sp_ref[1]
    h = jnp.maximum(stot * dinv + b_ref[...], 0.0)
    bt = batch_ref[0, 0]
    seg = lax.broadcasted_iota(jnp.int32, (64, _BLK), 0)
    mask = (seg == jnp.broadcast_to(bt[None, :], (64, _BLK))).astype(jnp.float32)
    sums_sc[...] += jnp.dot(mask, h, preferred_element_type=jnp.float32)
    cnts_sc[...] += jnp.sum(mask, axis=1, keepdims=True)

    @pl.when(i == pl.num_programs(0) - 1)
    def _():
        emb = sums_sc[...] / jnp.maximum(cnts_sc[...], 1.0)
        hh = (jnp.dot(emb, wh1a_ref[...], preferred_element_type=jnp.float32)
              + jnp.dot(md_ref[...], wh1b_ref[...],
                        preferred_element_type=jnp.float32)
              + bh1_ref[...])
        hh = jnp.maximum(hh, 0.0)
        out_ref[...] = jnp.dot(hh, wh2_ref[...],
                               preferred_element_type=jnp.float32) + bh2_ref[...]


def _tc_in(x, W, degt):
    return pl.pallas_call(
        _tc_in_body,
        grid=(_N // _BLK,),
        in_specs=[
            pl.BlockSpec((_BLK, 128), lambda i: (i, 0)),
            pl.BlockSpec((128, _HID), lambda i: (0, 0)),
            pl.BlockSpec((_BLK, 2), lambda i: (i, 0)),
        ],
        out_specs=pl.BlockSpec((_BLK, _HID), lambda i: (i, 0)),
        out_shape=jax.ShapeDtypeStruct((_N, _HID), jnp.float32),
    )(x, W, degt)


def _tc_mid(y, sp, degt, W, b):
    return pl.pallas_call(
        _tc_mid_body,
        grid=(_N // _BLK,),
        in_specs=[
            pl.BlockSpec((_BLK, _HID), lambda i: (i, 0)),
            pl.BlockSpec((_NC, _BLK, _HID), lambda i: (0, i, 0)),
            pl.BlockSpec((_BLK, 2), lambda i: (i, 0)),
            pl.BlockSpec((_HID, _HID), lambda i: (0, 0)),
            pl.BlockSpec((_HID,), lambda i: (0,)),
        ],
        out_specs=pl.BlockSpec((_BLK, _HID), lambda i: (i, 0)),
        out_shape=jax.ShapeDtypeStruct((_N, _HID), jnp.float32),
    )(y, sp, degt, W, b)


def _tc_head(y, sp, degt, b, batch, md_pad, Wh1a, Wh1b_pad, bh1, Wh2, bh2):
    return pl.pallas_call(
        _tc_head_body,
        grid=(_N // _BLK,),
        in_specs=[
            pl.BlockSpec((_BLK, _HID), lambda i: (i, 0)),
            pl.BlockSpec((_NC, _BLK, _HID), lambda i: (0, i, 0)),
            pl.BlockSpec((_BLK, 2), lambda i: (i, 0)),
            pl.BlockSpec((_HID,), lambda i: (0,)),
            pl.BlockSpec((1, 1, _BLK), lambda i: (i, 0, 0)),
            pl.BlockSpec((64, 32), lambda i: (0, 0)),
            pl.BlockSpec((_HID, _HID), lambda i: (0, 0)),
            pl.BlockSpec((32, _HID), lambda i: (0, 0)),
            pl.BlockSpec((_HID,), lambda i: (0,)),
            pl.BlockSpec((_HID, 1), lambda i: (0, 0)),
            pl.BlockSpec((1, 1), lambda i: (0, 0)),
        ],
        out_specs=pl.BlockSpec((64, 1), lambda i: (0, 0)),
        out_shape=jax.ShapeDtypeStruct((64, 1), jnp.float32),
        scratch_shapes=[
            pltpu.VMEM((64, _HID), jnp.float32),
            pltpu.VMEM((64, 1), jnp.float32),
        ],
    )(y, sp, degt, b, batch, md_pad, Wh1a, Wh1b_pad, bh1, Wh2, bh2)


def kernel(x, edge_index, batch, metadata, W1, b1, W2, b2, W3, b3,
           Wh1, bh1, Wh2, bh2):
    src2 = edge_index[0].reshape(_NW * _NCHUNK, _CH)
    dst2 = edge_index[1].reshape(_NW * _NCHUNK, _CH)

    ones_d = jnp.ones((_CH, _DW), jnp.float32)
    zeros_d = jnp.zeros((_RPS, _DW), jnp.float32)
    zeros_h = jnp.zeros((_RPS, _HID), jnp.float32)

    degp = _sc_deg(dst2, ones_d, zeros_d)
    degt = degp[:, 0].reshape(_NC, _NPAD)[:, :_N].T

    y1 = _tc_in(x, W1, degt)
    s1 = _sc_agg(src2, dst2, y1, zeros_h).reshape(_NC, _NPAD, _HID)
    y2 = _tc_mid(y1, s1, degt, W2, b1)
    s2 = _sc_agg(src2, dst2, y2, zeros_h).reshape(_NC, _NPAD, _HID)
    y3 = _tc_mid(y2, s2, degt, W3, b2)
    s3 = _sc_agg(src2, dst2, y3, zeros_h).reshape(_NC, _NPAD, _HID)

    md_pad = jnp.pad(metadata, ((0, 0), (0, 32 - metadata.shape[1])))
    Wh1a = Wh1[:_HID]
    Wh1b_pad = jnp.pad(Wh1[_HID:], ((0, 32 - (Wh1.shape[0] - _HID)), (0, 0)))
    bh2r = bh2.reshape(1, 1)

    batch3 = batch.reshape(_N // _BLK, 1, _BLK)
    return _tc_head(y3, s3, degt, b3, batch3, md_pad, Wh1a, Wh1b_pad,
                    bh1, Wh2, bh2r)


# speedup vs baseline: 37.0367x; 1.0319x over previous
import functools as _functools

import jax
import jax.numpy as jnp
from jax import lax
from jax.experimental import pallas as pl
from jax.experimental.pallas import tpu as pltpu
from jax.experimental.pallas import tpu_sc as plsc

_N = 10000
_E = 320000
_HID = 64
_NC, _NS = 2, 16
_NW = _NC * _NS
_CH = 125
_NCHUNK = _E // (_NW * _CH)
_RPS = 1000
_NZS = _N // _RPS
_DW = 8
_BLK = 2000
_NBUF = 8


@_functools.lru_cache(maxsize=None)
def _sc_mesh():
    return plsc.VectorSubcoreMesh(
        core_axis_name="c", subcore_axis_name="s",
        num_cores=_NC, num_subcores=_NS)


def _deg_body(dst_hbm, ones_hbm, zeros_hbm, out_hbm, dst_v, ones_v, acc_sh):
    c = lax.axis_index("c")
    s = lax.axis_index("s")
    wid = c * _NS + s
    r0 = s * _RPS

    @pl.when(s < _NZS)
    def _():
        pltpu.sync_copy(zeros_hbm, acc_sh.at[pl.ds(r0, _RPS)])

    pltpu.sync_copy(ones_hbm, ones_v)
    pltpu.sync_copy(dst_hbm.at[pl.ds(wid * _NCHUNK, _NCHUNK)], dst_v)
    plsc.subcore_barrier()

    @pl.loop(0, _NCHUNK)
    def _(j):
        pltpu.sync_copy(ones_v, acc_sh.at[dst_v.at[j]], add=True)

    plsc.subcore_barrier()

    @pl.when(s < _NZS)
    def _():
        pltpu.sync_copy(acc_sh.at[pl.ds(r0, _RPS)],
                        out_hbm.at[pl.ds(c * _N + r0, _RPS)])


def _sc_deg(dst2, ones_d, zeros_d):
    return pl.kernel(
        _deg_body,
        out_type=jax.ShapeDtypeStruct((_NC * _N, _DW), jnp.float32),
        mesh=_sc_mesh(),
        scratch_types=[
            pltpu.VMEM((_NCHUNK, _CH), jnp.int32),
            pltpu.VMEM((_CH, _DW), jnp.float32),
            pltpu.VMEM_SHARED((_N, _DW), jnp.float32),
        ],
        compiler_params=pltpu.CompilerParams(use_tc_tiling_on_sc=False),
    )(dst2, ones_d, zeros_d)


def _agg_body(src_hbm, dst_hbm, table_hbm, zeros_hbm, out_hbm,
              src_v, dst_v, rows_v, acc_sh, *sems):
    semg = sems[:_NBUF]
    sems_ = sems[_NBUF:]
    c = lax.axis_index("c")
    s = lax.axis_index("s")
    wid = c * _NS + s
    r0 = s * _RPS

    @pl.when(s < _NZS)
    def _():
        pltpu.sync_copy(zeros_hbm, acc_sh.at[pl.ds(r0, _RPS)])

    pltpu.sync_copy(src_hbm.at[pl.ds(wid * _NCHUNK, _NCHUNK)], src_v)
    pltpu.sync_copy(dst_hbm.at[pl.ds(wid * _NCHUNK, _NCHUNK)], dst_v)
    plsc.subcore_barrier()

    for b in range(_NBUF):
        pltpu.async_copy(table_hbm.at[src_v.at[b]], rows_v[b], semg[b])

    @pl.loop(0, _NCHUNK, step=_NBUF)
    def _(j):
        for b in range(_NBUF):
            pltpu.make_async_copy(table_hbm.at[src_v.at[j + b]], rows_v[b],
                                  semg[b]).wait()
            pltpu.async_copy(rows_v[b], acc_sh.at[dst_v.at[j + b]], sems_[b],
                             add=True)
        for b in range(_NBUF):
            @pl.when(j + b + _NBUF < _NCHUNK)
            def _(b=b):
                pltpu.make_async_copy(rows_v[b], acc_sh.at[dst_v.at[j + b]],
                                      sems_[b]).wait()
                pltpu.async_copy(table_hbm.at[src_v.at[j + b + _NBUF]],
                                 rows_v[b], semg[b])

    for b in range(_NBUF):
        pltpu.make_async_copy(rows_v[b],
                              acc_sh.at[dst_v.at[_NCHUNK - _NBUF + b]],
                              sems_[b]).wait()

    plsc.subcore_barrier()

    @pl.when(s < _NZS)
    def _():
        pltpu.sync_copy(acc_sh.at[pl.ds(r0, _RPS)],
                        out_hbm.at[pl.ds(c * _N + r0, _RPS)])


def _sc_agg(src2, dst2, table, zeros_h):
    return pl.kernel(
        _agg_body,
        out_type=jax.ShapeDtypeStruct((_NC * _N, _HID), jnp.float32),
        mesh=_sc_mesh(),
        scratch_types=(
            [pltpu.VMEM((_NCHUNK, _CH), jnp.int32),
             pltpu.VMEM((_NCHUNK, _CH), jnp.int32),
             [pltpu.VMEM((_CH, _HID), jnp.float32) for _ in range(_NBUF)],
             pltpu.VMEM_SHARED((_N, _HID), jnp.float32)]
            + [pltpu.SemaphoreType.DMA] * (2 * _NBUF)
        ),
        compiler_params=pltpu.CompilerParams(use_tc_tiling_on_sc=False),
    )(src2, dst2, table, zeros_h)


def _dinv_of(deg_ref):
    deg = 1.0 + deg_ref[:, 0:1] + deg_ref[:, 1:2]
    return lax.rsqrt(deg)


def _tc_in_body(x_ref, w_ref, deg_ref, y_ref):
    dinv = _dinv_of(deg_ref)
    y_ref[...] = jnp.dot(x_ref[...], w_ref[...],
                         preferred_element_type=jnp.float32) * dinv


def _tc_mid_body(y_ref, sp_ref, deg_ref, w_ref, b_ref, o_ref):
    dinv = _dinv_of(deg_ref)
    stot = y_ref[...] + sp_ref[0] + sp_ref[1]
    h = jnp.maximum(stot * dinv + b_ref[...], 0.0)
    o_ref[...] = jnp.dot(h, w_ref[...],
                         preferred_element_type=jnp.float32) * dinv


def _tc_head_body(y_ref, sp_ref, deg_ref, b_ref, batch_ref, md_ref,
                  wh1a_ref, wh1b_ref, bh1_ref, wh2_ref, bh2_ref,
                  out_ref, sums_sc, cnts_sc):
    i = pl.program_id(0)

    @pl.when(i == 0)
    def _():
        sums_sc[...] = jnp.zeros_like(sums_sc)
        cnts_sc[...] = jnp.zeros_like(cnts_sc)

    dinv = _dinv_of(deg_ref)
    stot = y_ref[...] + sp_ref[0] + sp_ref[1]
    h = jnp.maximum(stot * dinv + b_ref[...], 0.0)
    bt = batch_ref[0, 0]
    seg = lax.broadcasted_iota(jnp.int32, (64, _BLK), 0)
    mask = (seg == jnp.broadcast_to(bt[None, :], (64, _BLK))).astype(jnp.float32)
    sums_sc[...] += jnp.dot(mask, h, preferred_element_type=jnp.float32)
    cnts_sc[...] += jnp.sum(mask, axis=1, keepdims=True)

    @pl.when(i == pl.num_programs(0) - 1)
    def _():
        emb = sums_sc[...] / jnp.maximum(cnts_sc[...], 1.0)
        hh = (jnp.dot(emb, wh1a_ref[...], preferred_element_type=jnp.float32)
              + jnp.dot(md_ref[...], wh1b_ref[...],
                        preferred_element_type=jnp.float32)
              + bh1_ref[...])
        hh = jnp.maximum(hh, 0.0)
        out_ref[...] = jnp.dot(hh, wh2_ref[...],
                               preferred_element_type=jnp.float32) + bh2_ref[...]


def _tc_in(x, W, degt):
    return pl.pallas_call(
        _tc_in_body,
        grid=(_N // _BLK,),
        in_specs=[
            pl.BlockSpec((_BLK, 128), lambda i: (i, 0)),
            pl.BlockSpec((128, _HID), lambda i: (0, 0)),
            pl.BlockSpec((_BLK, 2), lambda i: (i, 0)),
        ],
        out_specs=pl.BlockSpec((_BLK, _HID), lambda i: (i, 0)),
        out_shape=jax.ShapeDtypeStruct((_N, _HID), jnp.float32),
    )(x, W, degt)


def _tc_mid(y, sp, degt, W, b):
    return pl.pallas_call(
        _tc_mid_body,
        grid=(_N // _BLK,),
        in_specs=[
            pl.BlockSpec((_BLK, _HID), lambda i: (i, 0)),
            pl.BlockSpec((_NC, _BLK, _HID), lambda i: (0, i, 0)),
            pl.BlockSpec((_BLK, 2), lambda i: (i, 0)),
            pl.BlockSpec((_HID, _HID), lambda i: (0, 0)),
            pl.BlockSpec((_HID,), lambda i: (0,)),
        ],
        out_specs=pl.BlockSpec((_BLK, _HID), lambda i: (i, 0)),
        out_shape=jax.ShapeDtypeStruct((_N, _HID), jnp.float32),
    )(y, sp, degt, W, b)


def _tc_head(y, sp, degt, b, batch, md_pad, Wh1a, Wh1b_pad, bh1, Wh2, bh2):
    return pl.pallas_call(
        _tc_head_body,
        grid=(_N // _BLK,),
        in_specs=[
            pl.BlockSpec((_BLK, _HID), lambda i: (i, 0)),
            pl.BlockSpec((_NC, _BLK, _HID), lambda i: (0, i, 0)),
            pl.BlockSpec((_BLK, 2), lambda i: (i, 0)),
            pl.BlockSpec((_HID,), lambda i: (0,)),
            pl.BlockSpec((1, 1, _BLK), lambda i: (i, 0, 0)),
            pl.BlockSpec((64, 32), lambda i: (0, 0)),
            pl.BlockSpec((_HID, _HID), lambda i: (0, 0)),
            pl.BlockSpec((32, _HID), lambda i: (0, 0)),
            pl.BlockSpec((_HID,), lambda i: (0,)),
            pl.BlockSpec((_HID, 1), lambda i: (0, 0)),
            pl.BlockSpec((1, 1), lambda i: (0, 0)),
        ],
        out_specs=pl.BlockSpec((64, 1), lambda i: (0, 0)),
        out_shape=jax.ShapeDtypeStruct((64, 1), jnp.float32),
        scratch_shapes=[
            pltpu.VMEM((64, _HID), jnp.float32),
            pltpu.VMEM((64, 1), jnp.float32),
        ],
    )(y, sp, degt, b, batch, md_pad, Wh1a, Wh1b_pad, bh1, Wh2, bh2)


def kernel(x, edge_index, batch, metadata, W1, b1, W2, b2, W3, b3,
           Wh1, bh1, Wh2, bh2):
    src2 = edge_index[0].reshape(_NW * _NCHUNK, _CH)
    dst2 = edge_index[1].reshape(_NW * _NCHUNK, _CH)

    ones_d = jnp.ones((_CH, _DW), jnp.float32)
    zeros_d = jnp.zeros((_RPS, _DW), jnp.float32)
    zeros_h = jnp.zeros((_RPS, _HID), jnp.float32)

    degp = _sc_deg(dst2, ones_d, zeros_d)
    degt = degp[:, 0].reshape(_NC, _N).T

    y1 = _tc_in(x, W1, degt)
    s1 = _sc_agg(src2, dst2, y1, zeros_h).reshape(_NC, _N, _HID)
    y2 = _tc_mid(y1, s1, degt, W2, b1)
    s2 = _sc_agg(src2, dst2, y2, zeros_h).reshape(_NC, _N, _HID)
    y3 = _tc_mid(y2, s2, degt, W3, b2)
    s3 = _sc_agg(src2, dst2, y3, zeros_h).reshape(_NC, _N, _HID)

    md_pad = jnp.pad(metadata, ((0, 0), (0, 32 - metadata.shape[1])))
    Wh1a = Wh1[:_HID]
    Wh1b_pad = jnp.pad(Wh1[_HID:], ((0, 32 - (Wh1.shape[0] - _HID)), (0, 0)))
    bh2r = bh2.reshape(1, 1)

    batch3 = batch.reshape(_N // _BLK, 1, _BLK)
    return _tc_head(y3, s3, degt, b3, batch3, md_pad, Wh1a, Wh1b_pad,
                    bh1, Wh2, bh2r)


# speedup vs baseline: 37.9489x; 1.0246x over previous
import functools as _functools

import jax
import jax.numpy as jnp
from jax import lax
from jax.experimental import pallas as pl
from jax.experimental.pallas import tpu as pltpu
from jax.experimental.pallas import tpu_sc as plsc

_N = 10000
_E = 320000
_HID = 64
_NC, _NS = 2, 16
_NW = _NC * _NS
_CH = 125
_NCHUNK = _E // (_NW * _CH)
_RPS = 1000
_NZS = _N // _RPS
_DW = 8
_BLK = 2000
_NBUF = 8


@_functools.lru_cache(maxsize=None)
def _sc_mesh():
    return plsc.VectorSubcoreMesh(
        core_axis_name="c", subcore_axis_name="s",
        num_cores=_NC, num_subcores=_NS)


def _deg_body(ei_hbm, ones_hbm, zeros_hbm, out_hbm, dst_v, ones_v, acc_sh):
    c = lax.axis_index("c")
    s = lax.axis_index("s")
    wid = c * _NS + s
    r0 = s * _RPS

    @pl.when(s < _NZS)
    def _():
        pltpu.sync_copy(zeros_hbm, acc_sh.at[pl.ds(r0, _RPS)])

    pltpu.sync_copy(ones_hbm, ones_v)
    pltpu.sync_copy(ei_hbm.at[1].at[pl.ds(wid * _NCHUNK, _NCHUNK)], dst_v)
    plsc.subcore_barrier()

    @pl.loop(0, _NCHUNK)
    def _(j):
        pltpu.sync_copy(ones_v, acc_sh.at[dst_v.at[j]], add=True)

    plsc.subcore_barrier()

    @pl.when(s < _NZS)
    def _():
        pltpu.sync_copy(acc_sh.at[pl.ds(r0, _RPS)],
                        out_hbm.at[pl.ds(c * _N + r0, _RPS)])


def _sc_deg(ei3, ones_d, zeros_d):
    return pl.kernel(
        _deg_body,
        out_type=jax.ShapeDtypeStruct((_NC * _N, _DW), jnp.float32),
        mesh=_sc_mesh(),
        scratch_types=[
            pltpu.VMEM((_NCHUNK, _CH), jnp.int32),
            pltpu.VMEM((_CH, _DW), jnp.float32),
            pltpu.VMEM_SHARED((_N, _DW), jnp.float32),
        ],
        compiler_params=pltpu.CompilerParams(use_tc_tiling_on_sc=False),
    )(ei3, ones_d, zeros_d)


def _agg_body(ei_hbm, table_hbm, zeros_hbm, out_hbm,
              src_v, dst_v, rows_v, acc_sh, *sems):
    semg = sems[:_NBUF]
    sems_ = sems[_NBUF:]
    c = lax.axis_index("c")
    s = lax.axis_index("s")
    wid = c * _NS + s
    r0 = s * _RPS

    @pl.when(s < _NZS)
    def _():
        pltpu.sync_copy(zeros_hbm, acc_sh.at[pl.ds(r0, _RPS)])

    pltpu.sync_copy(ei_hbm.at[0].at[pl.ds(wid * _NCHUNK, _NCHUNK)], src_v)
    pltpu.sync_copy(ei_hbm.at[1].at[pl.ds(wid * _NCHUNK, _NCHUNK)], dst_v)
    plsc.subcore_barrier()

    for b in range(_NBUF):
        pltpu.async_copy(table_hbm.at[src_v.at[b]], rows_v[b], semg[b])

    @pl.loop(0, _NCHUNK, step=_NBUF)
    def _(j):
        for b in range(_NBUF):
            pltpu.make_async_copy(table_hbm.at[src_v.at[j + b]], rows_v[b],
                                  semg[b]).wait()
            pltpu.async_copy(rows_v[b], acc_sh.at[dst_v.at[j + b]], sems_[b],
                             add=True)
        for b in range(_NBUF):
            @pl.when(j + b + _NBUF < _NCHUNK)
            def _(b=b):
                pltpu.make_async_copy(rows_v[b], acc_sh.at[dst_v.at[j + b]],
                                      sems_[b]).wait()
                pltpu.async_copy(table_hbm.at[src_v.at[j + b + _NBUF]],
                                 rows_v[b], semg[b])

    for b in range(_NBUF):
        pltpu.make_async_copy(rows_v[b],
                              acc_sh.at[dst_v.at[_NCHUNK - _NBUF + b]],
                              sems_[b]).wait()

    plsc.subcore_barrier()

    @pl.when(s < _NZS)
    def _():
        pltpu.sync_copy(acc_sh.at[pl.ds(r0, _RPS)],
                        out_hbm.at[pl.ds(c * _N + r0, _RPS)])


def _sc_agg(ei3, table, zeros_h):
    return pl.kernel(
        _agg_body,
        out_type=jax.ShapeDtypeStruct((_NC * _N, _HID), jnp.float32),
        mesh=_sc_mesh(),
        scratch_types=(
            [pltpu.VMEM((_NCHUNK, _CH), jnp.int32),
             pltpu.VMEM((_NCHUNK, _CH), jnp.int32),
             [pltpu.VMEM((_CH, _HID), jnp.float32) for _ in range(_NBUF)],
             pltpu.VMEM_SHARED((_N, _HID), jnp.float32)]
            + [pltpu.SemaphoreType.DMA] * (2 * _NBUF)
        ),
        compiler_params=pltpu.CompilerParams(use_tc_tiling_on_sc=False),
    )(ei3, table, zeros_h)


def _dinv_of(deg_ref):
    deg = 1.0 + deg_ref[:, 0:1] + deg_ref[:, 1:2]
    return lax.rsqrt(deg)


def _tc_in_body(x_ref, w_ref, deg_ref, y_ref):
    dinv = _dinv_of(deg_ref)
    y_ref[...] = jnp.dot(x_ref[...], w_ref[...],
                         preferred_element_type=jnp.float32) * dinv


def _tc_mid_body(y_ref, sp_ref, deg_ref, w_ref, b_ref, o_ref):
    dinv = _dinv_of(deg_ref)
    stot = y_ref[...] + sp_ref[0] + sp_ref[1]
    h = jnp.maximum(stot * dinv + b_ref[...], 0.0)
    o_ref[...] = jnp.dot(h, w_ref[...],
                         preferred_element_type=jnp.float32) * dinv


def _tc_head_body(y_ref, sp_ref, deg_ref, b_ref, batch_ref, md_ref,
                  wh1a_ref, wh1b_ref, bh1_ref, wh2_ref, bh2_ref,
                  out_ref, sums_sc, cnts_sc):
    i = pl.program_id(0)

    @pl.when(i == 0)
    def _():
        sums_sc[...] = jnp.zeros_like(sums_sc)
        cnts_sc[...] = jnp.zeros_like(cnts_sc)

    dinv = _dinv_of(deg_ref)
    stot = y_ref[...] + sp_ref[0] + sp_ref[1]
    h = jnp.maximum(stot * dinv + b_ref[...], 0.0)
    bt = batch_ref[0, 0]
    seg = lax.broadcasted_iota(jnp.int32, (64, _BLK), 0)
    mask = (seg == jnp.broadcast_to(bt[None, :], (64, _BLK))).astype(jnp.float32)
    sums_sc[...] += jnp.dot(mask, h, preferred_element_type=jnp.float32)
    cnts_sc[...] += jnp.sum(mask, axis=1, keepdims=True)

    @pl.when(i == pl.num_programs(0) - 1)
    def _():
        emb = sums_sc[...] / jnp.maximum(cnts_sc[...], 1.0)
        hh = (jnp.dot(emb, wh1a_ref[...], preferred_element_type=jnp.float32)
              + jnp.dot(md_ref[...], wh1b_ref[...],
                        preferred_element_type=jnp.float32)
              + bh1_ref[...])
        hh = jnp.maximum(hh, 0.0)
        out_ref[...] = jnp.dot(hh, wh2_ref[...],
                               preferred_element_type=jnp.float32) + bh2_ref[...]


def _tc_in(x, W, degt):
    return pl.pallas_call(
        _tc_in_body,
        grid=(_N // _BLK,),
        in_specs=[
            pl.BlockSpec((_BLK, 128), lambda i: (i, 0)),
            pl.BlockSpec((128, _HID), lambda i: (0, 0)),
            pl.BlockSpec((_BLK, 2), lambda i: (i, 0)),
        ],
        out_specs=pl.BlockSpec((_BLK, _HID), lambda i: (i, 0)),
        out_shape=jax.ShapeDtypeStruct((_N, _HID), jnp.float32),
    )(x, W, degt)


def _tc_mid(y, sp, degt, W, b):
    return pl.pallas_call(
        _tc_mid_body,
        grid=(_N // _BLK,),
        in_specs=[
            pl.BlockSpec((_BLK, _HID), lambda i: (i, 0)),
            pl.BlockSpec((_NC, _BLK, _HID), lambda i: (0, i, 0)),
            pl.BlockSpec((_BLK, 2), lambda i: (i, 0)),
            pl.BlockSpec((_HID, _HID), lambda i: (0, 0)),
            pl.BlockSpec((_HID,), lambda i: (0,)),
        ],
        out_specs=pl.BlockSpec((_BLK, _HID), lambda i: (i, 0)),
        out_shape=jax.ShapeDtypeStruct((_N, _HID), jnp.float32),
    )(y, sp, degt, W, b)


def _tc_head(y, sp, degt, b, batch, md_pad, Wh1a, Wh1b_pad, bh1, Wh2, bh2):
    return pl.pallas_call(
        _tc_head_body,
        grid=(_N // _BLK,),
        in_specs=[
            pl.BlockSpec((_BLK, _HID), lambda i: (i, 0)),
            pl.BlockSpec((_NC, _BLK, _HID), lambda i: (0, i, 0)),
            pl.BlockSpec((_BLK, 2), lambda i: (i, 0)),
            pl.BlockSpec((_HID,), lambda i: (0,)),
            pl.BlockSpec((1, 1, _BLK), lambda i: (i, 0, 0)),
            pl.BlockSpec((64, 32), lambda i: (0, 0)),
            pl.BlockSpec((_HID, _HID), lambda i: (0, 0)),
            pl.BlockSpec((32, _HID), lambda i: (0, 0)),
            pl.BlockSpec((_HID,), lambda i: (0,)),
            pl.BlockSpec((_HID, 1), lambda i: (0, 0)),
            pl.BlockSpec((1, 1), lambda i: (0, 0)),
        ],
        out_specs=pl.BlockSpec((64, 1), lambda i: (0, 0)),
        out_shape=jax.ShapeDtypeStruct((64, 1), jnp.float32),
        scratch_shapes=[
            pltpu.VMEM((64, _HID), jnp.float32),
            pltpu.VMEM((64, 1), jnp.float32),
        ],
    )(y, sp, degt, b, batch, md_pad, Wh1a, Wh1b_pad, bh1, Wh2, bh2)


def kernel(x, edge_index, batch, metadata, W1, b1, W2, b2, W3, b3,
           Wh1, bh1, Wh2, bh2):
    ei3 = edge_index.reshape(2, _NW * _NCHUNK, _CH)

    ones_d = jnp.ones((_CH, _DW), jnp.float32)
    zeros_d = jnp.zeros((_RPS, _DW), jnp.float32)
    zeros_h = jnp.zeros((_RPS, _HID), jnp.float32)

    degp = _sc_deg(ei3, ones_d, zeros_d)
    degt = degp[:, 0].reshape(_NC, _N).T

    y1 = _tc_in(x, W1, degt)
    s1 = _sc_agg(ei3, y1, zeros_h).reshape(_NC, _N, _HID)
    y2 = _tc_mid(y1, s1, degt, W2, b1)
    s2 = _sc_agg(ei3, y2, zeros_h).reshape(_NC, _N, _HID)
    y3 = _tc_mid(y2, s2, degt, W3, b2)
    s3 = _sc_agg(ei3, y3, zeros_h).reshape(_NC, _N, _HID)

    md_pad = jnp.pad(metadata, ((0, 0), (0, 32 - metadata.shape[1])))
    Wh1a = Wh1[:_HID]
    Wh1b_pad = jnp.pad(Wh1[_HID:], ((0, 32 - (Wh1.shape[0] - _HID)), (0, 0)))
    bh2r = bh2.reshape(1, 1)

    batch3 = batch.reshape(_N // _BLK, 1, _BLK)
    return _tc_head(y3, s3, degt, b3, batch3, md_pad, Wh1a, Wh1b_pad,
                    bh1, Wh2, bh2r)


# speedup vs baseline: 41.4399x; 1.0920x over previous
import functools as _functools

import jax
import jax.numpy as jnp
from jax import lax
from jax.experimental import pallas as pl
from jax.experimental.pallas import tpu as pltpu
from jax.experimental.pallas import tpu_sc as plsc

_N = 10000
_E = 320000
_HID = 64
_NC, _NS = 2, 16
_NW = _NC * _NS
_CH = 125
_NCHUNK = _E // (_NW * _CH)
_RPS = 1000
_NZS = _N // _RPS
_DW = 8
_BLK = 2000
_NBUF = 8


@_functools.lru_cache(maxsize=None)
def _sc_mesh():
    return plsc.VectorSubcoreMesh(
        core_axis_name="c", subcore_axis_name="s",
        num_cores=_NC, num_subcores=_NS)


def _deg_body(ei_hbm, ones_hbm, zeros_hbm, out_hbm, dst_v, ones_v, acc_sh):
    c = lax.axis_index("c")
    s = lax.axis_index("s")
    wid = c * _NS + s
    r0 = s * _RPS

    @pl.when(s < _NZS)
    def _():
        pltpu.sync_copy(zeros_hbm, acc_sh.at[pl.ds(r0, _RPS)])

    pltpu.sync_copy(ones_hbm, ones_v)
    pltpu.sync_copy(ei_hbm.at[1].at[pl.ds(wid * _NCHUNK, _NCHUNK)], dst_v)
    plsc.subcore_barrier()

    @pl.loop(0, _NCHUNK)
    def _(j):
        pltpu.sync_copy(ones_v, acc_sh.at[dst_v.at[j]], add=True)

    plsc.subcore_barrier()

    @pl.when(s < _NZS)
    def _():
        pltpu.sync_copy(acc_sh.at[pl.ds(r0, _RPS)],
                        out_hbm.at[pl.ds(c * _N + r0, _RPS)])


def _sc_deg(ei3, ones_d, zeros_d):
    return pl.kernel(
        _deg_body,
        out_type=jax.ShapeDtypeStruct((_NC * _N, _DW), jnp.float32),
        mesh=_sc_mesh(),
        scratch_types=[
            pltpu.VMEM((_NCHUNK, _CH), jnp.int32),
            pltpu.VMEM((_CH, _DW), jnp.float32),
            pltpu.VMEM_SHARED((_N, _DW), jnp.float32),
        ],
        compiler_params=pltpu.CompilerParams(use_tc_tiling_on_sc=False),
    )(ei3, ones_d, zeros_d)


def _agg_body(ei_hbm, table_hbm, zeros_hbm, out_hbm,
              src_v, dst_v, rows_v, acc_sh, *sems):
    semg = sems[:_NBUF]
    sems_ = sems[_NBUF:]
    c = lax.axis_index("c")
    s = lax.axis_index("s")
    wid = c * _NS + s
    r0 = s * _RPS

    @pl.when(s < _NZS)
    def _():
        pltpu.sync_copy(zeros_hbm, acc_sh.at[pl.ds(r0, _RPS)])

    pltpu.sync_copy(ei_hbm.at[0].at[pl.ds(wid * _NCHUNK, _NCHUNK)], src_v)
    pltpu.sync_copy(ei_hbm.at[1].at[pl.ds(wid * _NCHUNK, _NCHUNK)], dst_v)
    plsc.subcore_barrier()

    for b in range(_NBUF):
        pltpu.async_copy(table_hbm.at[src_v.at[b]], rows_v[b], semg[b])

    @pl.loop(0, _NCHUNK, step=_NBUF)
    def _(j):
        for b in range(_NBUF):
            pltpu.make_async_copy(table_hbm.at[src_v.at[j + b]], rows_v[b],
                                  semg[b]).wait()
            pltpu.async_copy(rows_v[b], acc_sh.at[dst_v.at[j + b]], sems_[b],
                             add=True)
        for b in range(_NBUF):
            @pl.when(j + b + _NBUF < _NCHUNK)
            def _(b=b):
                pltpu.make_async_copy(rows_v[b], acc_sh.at[dst_v.at[j + b]],
                                      sems_[b]).wait()
                pltpu.async_copy(table_hbm.at[src_v.at[j + b + _NBUF]],
                                 rows_v[b], semg[b])

    for b in range(_NBUF):
        pltpu.make_async_copy(rows_v[b],
                              acc_sh.at[dst_v.at[_NCHUNK - _NBUF + b]],
                              sems_[b]).wait()

    plsc.subcore_barrier()

    @pl.when(s < _NZS)
    def _():
        pltpu.sync_copy(acc_sh.at[pl.ds(r0, _RPS)],
                        out_hbm.at[c, pl.ds(r0, _RPS), pl.ds(0, _HID)])


def _sc_agg(ei3, table, zeros_h):
    return pl.kernel(
        _agg_body,
        out_type=jax.ShapeDtypeStruct((_NC, _N, 2 * _HID), jnp.float32),
        mesh=_sc_mesh(),
        scratch_types=(
            [pltpu.VMEM((_NCHUNK, _CH), jnp.int32),
             pltpu.VMEM((_NCHUNK, _CH), jnp.int32),
             [pltpu.VMEM((_CH, _HID), jnp.float32) for _ in range(_NBUF)],
             pltpu.VMEM_SHARED((_N, _HID), jnp.float32)]
            + [pltpu.SemaphoreType.DMA] * (2 * _NBUF)
        ),
        compiler_params=pltpu.CompilerParams(use_tc_tiling_on_sc=False),
    )(ei3, table, zeros_h)


def _dinv_of(deg_ref):
    deg = 1.0 + deg_ref[:, 0:1] + deg_ref[:, 1:2]
    return lax.rsqrt(deg)


def _tc_in_body(x_ref, w_ref, deg_ref, y_ref):
    dinv = _dinv_of(deg_ref)
    y_ref[...] = jnp.dot(x_ref[...], w_ref[...],
                         preferred_element_type=jnp.float32) * dinv


def _tc_mid_body(y_ref, sp_ref, deg_ref, w_ref, b_ref, o_ref):
    dinv = _dinv_of(deg_ref)
    stot = y_ref[...] + sp_ref[0, :, :_HID] + sp_ref[1, :, :_HID]
    h = jnp.maximum(stot * dinv + b_ref[...], 0.0)
    o_ref[...] = jnp.dot(h, w_ref[...],
                         preferred_element_type=jnp.float32) * dinv


def _tc_head_body(y_ref, sp_ref, deg_ref, b_ref, batch_ref, md_ref,
                  wh1a_ref, wh1b_ref, bh1_ref, wh2_ref, bh2_ref,
                  out_ref, sums_sc, cnts_sc):
    i = pl.program_id(0)

    @pl.when(i == 0)
    def _():
        sums_sc[...] = jnp.zeros_like(sums_sc)
        cnts_sc[...] = jnp.zeros_like(cnts_sc)

    dinv = _dinv_of(deg_ref)
    stot = y_ref[...] + sp_ref[0, :, :_HID] + sp_ref[1, :, :_HID]
    h = jnp.maximum(stot * dinv + b_ref[...], 0.0)
    bt = batch_ref[0, 0]
    seg = lax.broadcasted_iota(jnp.int32, (64, _BLK), 0)
    mask = (seg == jnp.broadcast_to(bt[None, :], (64, _BLK))).astype(jnp.float32)
    sums_sc[...] += jnp.dot(mask, h, preferred_element_type=jnp.float32)
    cnts_sc[...] += jnp.sum(mask, axis=1, keepdims=True)

    @pl.when(i == pl.num_programs(0) - 1)
    def _():
        emb = sums_sc[...] / jnp.maximum(cnts_sc[...], 1.0)
        hh = (jnp.dot(emb, wh1a_ref[...], preferred_element_type=jnp.float32)
              + jnp.dot(md_ref[...], wh1b_ref[...],
                        preferred_element_type=jnp.float32)
              + bh1_ref[...])
        hh = jnp.maximum(hh, 0.0)
        out_ref[...] = jnp.dot(hh, wh2_ref[...],
                               preferred_element_type=jnp.float32) + bh2_ref[...]


def _tc_in(x, W, degt):
    return pl.pallas_call(
        _tc_in_body,
        grid=(_N // _BLK,),
        in_specs=[
            pl.BlockSpec((_BLK, 128), lambda i: (i, 0)),
            pl.BlockSpec((128, _HID), lambda i: (0, 0)),
            pl.BlockSpec((_BLK, 2), lambda i: (i, 0)),
        ],
        out_specs=pl.BlockSpec((_BLK, _HID), lambda i: (i, 0)),
        out_shape=jax.ShapeDtypeStruct((_N, _HID), jnp.float32),
    )(x, W, degt)


def _tc_mid(y, sp, degt, W, b):
    return pl.pallas_call(
        _tc_mid_body,
        grid=(_N // _BLK,),
        in_specs=[
            pl.BlockSpec((_BLK, _HID), lambda i: (i, 0)),
            pl.BlockSpec((_NC, _BLK, 2 * _HID), lambda i: (0, i, 0)),
            pl.BlockSpec((_BLK, 2), lambda i: (i, 0)),
            pl.BlockSpec((_HID, _HID), lambda i: (0, 0)),
            pl.BlockSpec((_HID,), lambda i: (0,)),
        ],
        out_specs=pl.BlockSpec((_BLK, _HID), lambda i: (i, 0)),
        out_shape=jax.ShapeDtypeStruct((_N, _HID), jnp.float32),
    )(y, sp, degt, W, b)


def _tc_head(y, sp, degt, b, batch, md_pad, Wh1a, Wh1b_pad, bh1, Wh2, bh2):
    return pl.pallas_call(
        _tc_head_body,
        grid=(_N // _BLK,),
        in_specs=[
            pl.BlockSpec((_BLK, _HID), lambda i: (i, 0)),
            pl.BlockSpec((_NC, _BLK, 2 * _HID), lambda i: (0, i, 0)),
            pl.BlockSpec((_BLK, 2), lambda i: (i, 0)),
            pl.BlockSpec((_HID,), lambda i: (0,)),
            pl.BlockSpec((1, 1, _BLK), lambda i: (i, 0, 0)),
            pl.BlockSpec((64, 32), lambda i: (0, 0)),
            pl.BlockSpec((_HID, _HID), lambda i: (0, 0)),
            pl.BlockSpec((32, _HID), lambda i: (0, 0)),
            pl.BlockSpec((_HID,), lambda i: (0,)),
            pl.BlockSpec((_HID, 1), lambda i: (0, 0)),
            pl.BlockSpec((1, 1), lambda i: (0, 0)),
        ],
        out_specs=pl.BlockSpec((64, 1), lambda i: (0, 0)),
        out_shape=jax.ShapeDtypeStruct((64, 1), jnp.float32),
        scratch_shapes=[
            pltpu.VMEM((64, _HID), jnp.float32),
            pltpu.VMEM((64, 1), jnp.float32),
        ],
    )(y, sp, degt, b, batch, md_pad, Wh1a, Wh1b_pad, bh1, Wh2, bh2)


def kernel(x, edge_index, batch, metadata, W1, b1, W2, b2, W3, b3,
           Wh1, bh1, Wh2, bh2):
    ei3 = edge_index.reshape(2, _NW * _NCHUNK, _CH)

    ones_d = jnp.ones((_CH, _DW), jnp.float32)
    zeros_d = jnp.zeros((_RPS, _DW), jnp.float32)
    zeros_h = jnp.zeros((_RPS, _HID), jnp.float32)

    degp = _sc_deg(ei3, ones_d, zeros_d)
    degt = degp[:, 0].reshape(_NC, _N).T

    y1 = _tc_in(x, W1, degt)
    s1 = _sc_agg(ei3, y1, zeros_h)
    y2 = _tc_mid(y1, s1, degt, W2, b1)
    s2 = _sc_agg(ei3, y2, zeros_h)
    y3 = _tc_mid(y2, s2, degt, W3, b2)
    s3 = _sc_agg(ei3, y3, zeros_h)

    md_pad = jnp.pad(metadata, ((0, 0), (0, 32 - metadata.shape[1])))
    Wh1a = Wh1[:_HID]
    Wh1b_pad = jnp.pad(Wh1[_HID:], ((0, 32 - (Wh1.shape[0] - _HID)), (0, 0)))
    bh2r = bh2.reshape(1, 1)

    batch3 = batch.reshape(_N // _BLK, 1, _BLK)
    return _tc_head(y3, s3, degt, b3, batch3, md_pad, Wh1a, Wh1b_pad,
                    bh1, Wh2, bh2r)


# speedup vs baseline: 43.5400x; 1.0507x over previous
import functools as _functools

import jax
import jax.numpy as jnp
from jax import lax
from jax.experimental import pallas as pl
from jax.experimental.pallas import tpu as pltpu
from jax.experimental.pallas import tpu_sc as plsc

_N = 10000
_E = 320000
_HID = 64
_NC, _NS = 2, 16
_NW = _NC * _NS
_CH = 125
_NCHUNK = _E // (_NW * _CH)
_RPS = 1000
_NZS = _N // _RPS
_DW = 8
_BLK = 2000
_NBUF = 8


@_functools.lru_cache(maxsize=None)
def _sc_mesh():
    return plsc.VectorSubcoreMesh(
        core_axis_name="c", subcore_axis_name="s",
        num_cores=_NC, num_subcores=_NS)


def _deg_body(ei_hbm, ones_hbm, zeros_hbm, out_hbm, dst_v, ones_v, acc_sh):
    c = lax.axis_index("c")
    s = lax.axis_index("s")
    wid = c * _NS + s
    r0 = s * _RPS

    @pl.when(s < _NZS)
    def _():
        pltpu.sync_copy(zeros_hbm, acc_sh.at[pl.ds(r0, _RPS)])

    pltpu.sync_copy(ones_hbm, ones_v)
    pltpu.sync_copy(ei_hbm.at[1].at[pl.ds(wid * _NCHUNK, _NCHUNK)], dst_v)
    plsc.subcore_barrier()

    @pl.loop(0, _NCHUNK)
    def _(j):
        pltpu.sync_copy(ones_v, acc_sh.at[dst_v.at[j]], add=True)

    plsc.subcore_barrier()

    @pl.when(s < _NZS)
    def _():
        pltpu.sync_copy(acc_sh.at[pl.ds(r0, _RPS)],
                        out_hbm.at[pl.ds(c * _N + r0, _RPS)])


def _sc_deg(ei3, ones_d, zeros_d):
    return pl.kernel(
        _deg_body,
        out_type=jax.ShapeDtypeStruct((_NC * _N, _DW), jnp.float32),
        mesh=_sc_mesh(),
        scratch_types=[
            pltpu.VMEM((_NCHUNK, _CH), jnp.int32),
            pltpu.VMEM((_CH, _DW), jnp.float32),
            pltpu.VMEM_SHARED((_N, _DW), jnp.float32),
        ],
        compiler_params=pltpu.CompilerParams(use_tc_tiling_on_sc=False),
    )(ei3, ones_d, zeros_d)


def _agg_body(ei_hbm, srcx2_hbm, table_hbm, zeros_hbm, out_hbm,
              src_v, dst_v, rows_v, acc_sh, *sems):
    semg = sems[:_NBUF]
    sems_ = sems[_NBUF:]
    c = lax.axis_index("c")
    s = lax.axis_index("s")
    wid = c * _NS + s
    r0 = s * _RPS

    @pl.when(s < _NZS)
    def _():
        pltpu.sync_copy(zeros_hbm, acc_sh.at[pl.ds(r0, _RPS)])

    pltpu.sync_copy(srcx2_hbm.at[pl.ds(wid * _NCHUNK, _NCHUNK)], src_v)
    pltpu.sync_copy(ei_hbm.at[1].at[pl.ds(wid * _NCHUNK, _NCHUNK)], dst_v)
    plsc.subcore_barrier()

    for b in range(_NBUF):
        pltpu.async_copy(table_hbm.at[src_v.at[b]], rows_v[b], semg[b])

    @pl.loop(0, _NCHUNK, step=_NBUF)
    def _(j):
        for b in range(_NBUF):
            pltpu.make_async_copy(table_hbm.at[src_v.at[j + b]], rows_v[b],
                                  semg[b]).wait()
            pltpu.async_copy(rows_v[b], acc_sh.at[dst_v.at[j + b]], sems_[b],
                             add=True)
        for b in range(_NBUF):
            @pl.when(j + b + _NBUF < _NCHUNK)
            def _(b=b):
                pltpu.make_async_copy(rows_v[b], acc_sh.at[dst_v.at[j + b]],
                                      sems_[b]).wait()
                pltpu.async_copy(table_hbm.at[src_v.at[j + b + _NBUF]],
                                 rows_v[b], semg[b])

    for b in range(_NBUF):
        pltpu.make_async_copy(rows_v[b],
                              acc_sh.at[dst_v.at[_NCHUNK - _NBUF + b]],
                              sems_[b]).wait()

    plsc.subcore_barrier()

    @pl.when(s < _NZS)
    def _():
        pltpu.sync_copy(acc_sh.at[pl.ds(r0, _RPS)],
                        out_hbm.at[c, pl.ds(r0, _RPS), pl.ds(0, _HID)])


def _sc_agg(ei3, srcx2, table, zeros_h):
    return pl.kernel(
        _agg_body,
        out_type=jax.ShapeDtypeStruct((_NC, _N, 2 * _HID), jnp.float32),
        mesh=_sc_mesh(),
        scratch_types=(
            [pltpu.VMEM((_NCHUNK, _CH), jnp.int32),
             pltpu.VMEM((_NCHUNK, _CH), jnp.int32),
             [pltpu.VMEM((_CH, _HID), jnp.float32) for _ in range(_NBUF)],
             pltpu.VMEM_SHARED((_N, _HID), jnp.float32)]
            + [pltpu.SemaphoreType.DMA] * (2 * _NBUF)
        ),
        compiler_params=pltpu.CompilerParams(use_tc_tiling_on_sc=False),
    )(ei3, srcx2, table, zeros_h)


def _dinv_of(deg_ref):
    deg = 1.0 + deg_ref[:, 0:1] + deg_ref[:, 1:2]
    return lax.rsqrt(deg)


def _tc_in_body(x_ref, w_ref, deg_ref, y_ref):
    dinv = _dinv_of(deg_ref)
    y_ref[:, :_HID] = jnp.dot(x_ref[...], w_ref[...],
                              preferred_element_type=jnp.float32) * dinv


def _tc_mid_body(y_ref, sp_ref, deg_ref, w_ref, b_ref, o_ref):
    dinv = _dinv_of(deg_ref)
    stot = y_ref[:, :_HID] + sp_ref[0, :, :_HID] + sp_ref[1, :, :_HID]
    h = jnp.maximum(stot * dinv + b_ref[...], 0.0)
    o_ref[:, :_HID] = jnp.dot(h, w_ref[...],
                              preferred_element_type=jnp.float32) * dinv


def _tc_head_body(y_ref, sp_ref, deg_ref, b_ref, batch_ref, md_ref,
                  wh1a_ref, wh1b_ref, bh1_ref, wh2_ref, bh2_ref,
                  out_ref, sums_sc, cnts_sc):
    i = pl.program_id(0)

    @pl.when(i == 0)
    def _():
        sums_sc[...] = jnp.zeros_like(sums_sc)
        cnts_sc[...] = jnp.zeros_like(cnts_sc)

    dinv = _dinv_of(deg_ref)
    stot = y_ref[:, :_HID] + sp_ref[0, :, :_HID] + sp_ref[1, :, :_HID]
    h = jnp.maximum(stot * dinv + b_ref[...], 0.0)
    bt = batch_ref[0, 0]
    seg = lax.broadcasted_iota(jnp.int32, (64, _BLK), 0)
    mask = (seg == jnp.broadcast_to(bt[None, :], (64, _BLK))).astype(jnp.float32)
    sums_sc[...] += jnp.dot(mask, h, preferred_element_type=jnp.float32)
    cnts_sc[...] += jnp.sum(mask, axis=1, keepdims=True)

    @pl.when(i == pl.num_programs(0) - 1)
    def _():
        emb = sums_sc[...] / jnp.maximum(cnts_sc[...], 1.0)
        hh = (jnp.dot(emb, wh1a_ref[...], preferred_element_type=jnp.float32)
              + jnp.dot(md_ref[...], wh1b_ref[...],
                        preferred_element_type=jnp.float32)
              + bh1_ref[...])
        hh = jnp.maximum(hh, 0.0)
        out_ref[...] = jnp.dot(hh, wh2_ref[...],
                               preferred_element_type=jnp.float32) + bh2_ref[...]


def _tc_in(x, W, degt):
    return pl.pallas_call(
        _tc_in_body,
        grid=(_N // _BLK,),
        in_specs=[
            pl.BlockSpec((_BLK, 128), lambda i: (i, 0)),
            pl.BlockSpec((128, _HID), lambda i: (0, 0)),
            pl.BlockSpec((_BLK, 2), lambda i: (i, 0)),
        ],
        out_specs=pl.BlockSpec((_BLK, 2 * _HID), lambda i: (i, 0)),
        out_shape=jax.ShapeDtypeStruct((_N, 2 * _HID), jnp.float32),
    )(x, W, degt)


def _tc_mid(y, sp, degt, W, b):
    return pl.pallas_call(
        _tc_mid_body,
        grid=(_N // _BLK,),
        in_specs=[
            pl.BlockSpec((_BLK, 2 * _HID), lambda i: (i, 0)),
            pl.BlockSpec((_NC, _BLK, 2 * _HID), lambda i: (0, i, 0)),
            pl.BlockSpec((_BLK, 2), lambda i: (i, 0)),
            pl.BlockSpec((_HID, _HID), lambda i: (0, 0)),
            pl.BlockSpec((_HID,), lambda i: (0,)),
        ],
        out_specs=pl.BlockSpec((_BLK, 2 * _HID), lambda i: (i, 0)),
        out_shape=jax.ShapeDtypeStruct((_N, 2 * _HID), jnp.float32),
    )(y, sp, degt, W, b)


def _tc_head(y, sp, degt, b, batch, md_pad, Wh1a, Wh1b_pad, bh1, Wh2, bh2):
    return pl.pallas_call(
        _tc_head_body,
        grid=(_N // _BLK,),
        in_specs=[
            pl.BlockSpec((_BLK, 2 * _HID), lambda i: (i, 0)),
            pl.BlockSpec((_NC, _BLK, 2 * _HID), lambda i: (0, i, 0)),
            pl.BlockSpec((_BLK, 2), lambda i: (i, 0)),
            pl.BlockSpec((_HID,), lambda i: (0,)),
            pl.BlockSpec((1, 1, _BLK), lambda i: (i, 0, 0)),
            pl.BlockSpec((64, 32), lambda i: (0, 0)),
            pl.BlockSpec((_HID, _HID), lambda i: (0, 0)),
            pl.BlockSpec((32, _HID), lambda i: (0, 0)),
            pl.BlockSpec((_HID,), lambda i: (0,)),
            pl.BlockSpec((_HID, 1), lambda i: (0, 0)),
            pl.BlockSpec((1, 1), lambda i: (0, 0)),
        ],
        out_specs=pl.BlockSpec((64, 1), lambda i: (0, 0)),
        out_shape=jax.ShapeDtypeStruct((64, 1), jnp.float32),
        scratch_shapes=[
            pltpu.VMEM((64, _HID), jnp.float32),
            pltpu.VMEM((64, 1), jnp.float32),
        ],
    )(y, sp, degt, b, batch, md_pad, Wh1a, Wh1b_pad, bh1, Wh2, bh2)


def kernel(x, edge_index, batch, metadata, W1, b1, W2, b2, W3, b3,
           Wh1, bh1, Wh2, bh2):
    ei3 = edge_index.reshape(2, _NW * _NCHUNK, _CH)
    srcx2 = ei3[0] * 2

    ones_d = jnp.ones((_CH, _DW), jnp.float32)
    zeros_d = jnp.zeros((_RPS, _DW), jnp.float32)
    zeros_h = jnp.zeros((_RPS, _HID), jnp.float32)

    degp = _sc_deg(ei3, ones_d, zeros_d)
    degt = degp[:, 0].reshape(_NC, _N).T

    y1 = _tc_in(x, W1, degt)
    s1 = _sc_agg(ei3, srcx2, y1.reshape(2 * _N, _HID), zeros_h)
    y2 = _tc_mid(y1, s1, degt, W2, b1)
    s2 = _sc_agg(ei3, srcx2, y2.reshape(2 * _N, _HID), zeros_h)
    y3 = _tc_mid(y2, s2, degt, W3, b2)
    s3 = _sc_agg(ei3, srcx2, y3.reshape(2 * _N, _HID), zeros_h)

    md_pad = jnp.pad(metadata, ((0, 0), (0, 32 - metadata.shape[1])))
    Wh1a = Wh1[:_HID]
    Wh1b_pad = jnp.pad(Wh1[_HID:], ((0, 32 - (Wh1.shape[0] - _HID)), (0, 0)))
    bh2r = bh2.reshape(1, 1)

    batch3 = batch.reshape(_N // _BLK, 1, _BLK)
    return _tc_head(y3, s3, degt, b3, batch3, md_pad, Wh1a, Wh1b_pad,
                    bh1, Wh2, bh2r)
